# Initial kernel scaffold; baseline (speedup 1.0000x reference)
import jax
import jax.numpy as jnp
from jax.experimental import pallas as pl


def kernel(x, W_gate, W_up, W_down):
    raise NotImplementedError("write your pallas kernel here")


# speedup vs baseline: 1.8721x; 1.8721x over previous
import functools

import jax
import jax.numpy as jnp
from jax.experimental import pallas as pl
from jax.experimental.pallas import tpu as pltpu

D = 2048
NE = 8
K = 2
ED = 8192
KE = 512
NSTEPS = ED // KE

_NT = (((1,), (1,)), ((), ()))


def _moe_kernel(x_ref, wg_ref, wup_ref, wdn_ref, out_ref, scale_ref):
    ke = pl.program_id(0)

    @pl.when(ke == 0)
    def _gate():
        logits = jax.lax.dot_general(
            x_ref[...], wg_ref[...], _NT, preferred_element_type=jnp.float32
        )
        m = jnp.max(logits, axis=1, keepdims=True)
        e = jnp.exp(logits - m)
        se = jnp.sum(e, axis=1, keepdims=True)
        m1 = jnp.max(e, axis=1, keepdims=True)
        a1 = jnp.argmax(e, axis=1, keepdims=True)
        lane = jax.lax.broadcasted_iota(jnp.int32, e.shape, 1)
        m2 = jnp.max(jnp.where(lane == a1, -jnp.inf, e), axis=1, keepdims=True)
        s = (m1 + m2) / se
        scale_ref[...] = s / (s + 1e-9)
        out_ref[...] = jnp.zeros_like(out_ref)

    h = jax.lax.dot_general(
        x_ref[...], wup_ref[...], _NT, preferred_element_type=jnp.float32
    )
    h = (h * jax.lax.logistic(h)).astype(jnp.bfloat16)
    contrib = jax.lax.dot_general(
        h, wdn_ref[...], _NT, preferred_element_type=jnp.float32
    )

    @pl.when(ke < NSTEPS - 1)
    def _acc():
        out_ref[...] += contrib

    @pl.when(ke == NSTEPS - 1)
    def _final():
        out_ref[...] = (out_ref[...] + contrib) * scale_ref[...]


@jax.jit
def kernel(x, W_gate, W_up, W_down):
    B, S, Dm = x.shape
    T = B * S
    xb = x.reshape(T, Dm).astype(jnp.bfloat16)
    wg = W_gate.astype(jnp.bfloat16)
    wup = W_up.astype(jnp.bfloat16)
    wdn = W_down.astype(jnp.bfloat16)

    out = pl.pallas_call(
        _moe_kernel,
        grid=(NSTEPS,),
        in_specs=[
            pl.BlockSpec((T, Dm), lambda ke: (0, 0)),
            pl.BlockSpec((NE, Dm), lambda ke: (0, 0)),
            pl.BlockSpec((KE, Dm), lambda ke: (ke, 0)),
            pl.BlockSpec((Dm, KE), lambda ke: (0, ke)),
        ],
        out_specs=pl.BlockSpec((T, Dm), lambda ke: (0, 0)),
        out_shape=jax.ShapeDtypeStruct((T, Dm), jnp.float32),
        scratch_shapes=[pltpu.VMEM((T, 1), jnp.float32)],
        compiler_params=pltpu.CompilerParams(
            dimension_semantics=("arbitrary",),
        ),
    )(xb, wg, wup, wdn)
    return out.reshape(B, S, Dm)
